# baseline (speedup 1.0000x reference)
import jax
import jax.numpy as jnp
from jax import lax
from jax.experimental import pallas as pl
from jax.experimental.pallas import tpu as pltpu
from jax.experimental.pallas import tpu_sc as plsc

N = 10000
E = 320000
D_IN = 128
D_H = 64
N_COMM = 10

NC = 2
NS = 16
NW = NC * NS

B = 128
NCHUNK = E // B
CPT = NCHUNK // NW
NEXTRA = NCHUNK - CPT * NW

RPT = N // NS
ZB = 25
DW = 16

_mesh = plsc.VectorSubcoreMesh(core_axis_name="c", subcore_axis_name="s")
_sc_params = pltpu.CompilerParams(use_tc_tiling_on_sc=False)


def _rsqrt_newton(x):
    y = jnp.full((16,), 0.75, jnp.float32)
    for t in (4.0, 16.0, 64.0, 256.0, 1024.0, 4096.0, 16384.0, 65536.0,
              262144.0):
        y = jnp.where(x >= t, y * 0.5, y)
    half = jnp.full((16,), 0.5, jnp.float32)
    th = jnp.full((16,), 1.5, jnp.float32)
    for _ in range(6):
        y = y * (th - half * x * y * y)
    return y


def _zero_fill(buf, rows, width):
    def zf(i, _):
        for j in range(width // 16):
            buf[i, pl.ds(j * 16, 16)] = jnp.zeros((16,), jnp.float32)
        return 0
    lax.fori_loop(0, rows, zf, 0)


def _load_idx(idxI, idx_v, wid):
    pltpu.sync_copy(idxI.at[pl.ds(wid * CPT, CPT)], idx_v.at[pl.ds(0, CPT)])

    @pl.when(wid < NEXTRA)
    def _extra():
        pltpu.sync_copy(idxI.at[pl.ds(NW * CPT + wid, 1)],
                        idx_v.at[pl.ds(CPT, 1)])


def _agg1_body(hw, srcI, dstI, out, dego, src_v, dst_v, rows_v, zb_v, ones_v,
               hb_v, db_v, acc_sh, hws_sh, deg_sh, g0):
    cid = lax.axis_index("c")
    sid = lax.axis_index("s")
    wid = cid * NS + sid
    base = sid * RPT

    _zero_fill(zb_v, ZB, D_H)
    _zero_fill(db_v, ZB, DW)
    for q in range(RPT // ZB):
        pltpu.sync_copy(zb_v, acc_sh.at[pl.ds(base + q * ZB, ZB)])
        pltpu.sync_copy(db_v, deg_sh.at[pl.ds(base + q * ZB, ZB)])

    def of(i, _):
        ones_v[i, :] = jnp.full((16,), 1.0, jnp.float32)
        return 0
    lax.fori_loop(0, B, of, 0)

    pltpu.sync_copy(hw.at[pl.ds(base, RPT)], hws_sh.at[pl.ds(base, RPT)])
    _load_idx(srcI, src_v, wid)
    _load_idx(dstI, dst_v, wid)
    plsc.subcore_barrier()

    def dchunk(c, _):
        pltpu.sync_copy(ones_v, deg_sh.at[dst_v.at[c]], add=True)
        return 0
    lax.fori_loop(0, CPT, dchunk, 0)

    @pl.when(wid < NEXTRA)
    def _dextra():
        pltpu.sync_copy(ones_v, deg_sh.at[dst_v.at[CPT]], add=True)
    plsc.subcore_barrier()

    def sblock(q, _):
        r0 = base + q * ZB
        pltpu.sync_copy(hws_sh.at[pl.ds(r0, ZB)], hb_v)
        pltpu.sync_copy(deg_sh.at[pl.ds(r0, ZB)], db_v)

        def srow(i, _):
            dv = _rsqrt_newton(db_v[i, :] + 1.0)
            for j in range(D_H // 16):
                hb_v[i, pl.ds(j * 16, 16)] = (
                    hb_v[i, pl.ds(j * 16, 16)] * dv)
            return 0
        lax.fori_loop(0, ZB, srow, 0)
        pltpu.sync_copy(hb_v, hws_sh.at[pl.ds(r0, ZB)])
        return 0
    lax.fori_loop(0, RPT // ZB, sblock, 0)
    plsc.subcore_barrier()

    def chunk(c, _):
        pltpu.async_copy(hws_sh.at[src_v.at[c]], rows_v, g0).wait()
        pltpu.sync_copy(rows_v, acc_sh.at[dst_v.at[c]], add=True)
        return 0
    lax.fori_loop(0, CPT, chunk, 0)

    @pl.when(wid < NEXTRA)
    def _gextra():
        pltpu.async_copy(hws_sh.at[src_v.at[CPT]], rows_v, g0).wait()
        pltpu.sync_copy(rows_v, acc_sh.at[dst_v.at[CPT]], add=True)
    plsc.subcore_barrier()

    pltpu.sync_copy(acc_sh.at[pl.ds(base, RPT)],
                    out.at[cid, pl.ds(base, RPT)])
    pltpu.sync_copy(deg_sh.at[pl.ds(base, RPT)],
                    dego.at[cid, pl.ds(base, RPT)])


_agg1_call = pl.kernel(
    _agg1_body,
    out_type=[
        jax.ShapeDtypeStruct((NC, N, D_H), jnp.float32),
        jax.ShapeDtypeStruct((NC, N, DW), jnp.float32),
    ],
    mesh=_mesh,
    compiler_params=_sc_params,
    scratch_types=[
        pltpu.VMEM((CPT + 1, B), jnp.int32),
        pltpu.VMEM((CPT + 1, B), jnp.int32),
        pltpu.VMEM((B, D_H), jnp.float32),
        pltpu.VMEM((ZB, D_H), jnp.float32),
        pltpu.VMEM((B, DW), jnp.float32),
        pltpu.VMEM((ZB, D_H), jnp.float32),
        pltpu.VMEM((ZB, DW), jnp.float32),
        pltpu.VMEM_SHARED((N, D_H), jnp.float32),
        pltpu.VMEM_SHARED((N, D_H), jnp.float32),
        pltpu.VMEM_SHARED((N, DW), jnp.float32),
        pltpu.SemaphoreType.DMA,
    ],
)


def _agg2_body(hws, srcI, dstI, out, src_v, dst_v, rows_v, zb_v, acc_sh,
               hws_sh, g0):
    cid = lax.axis_index("c")
    sid = lax.axis_index("s")
    wid = cid * NS + sid
    base = sid * RPT

    _zero_fill(zb_v, ZB, D_H)
    for q in range(RPT // ZB):
        pltpu.sync_copy(zb_v, acc_sh.at[pl.ds(base + q * ZB, ZB)])
    pltpu.sync_copy(hws.at[pl.ds(base, RPT)], hws_sh.at[pl.ds(base, RPT)])
    _load_idx(srcI, src_v, wid)
    _load_idx(dstI, dst_v, wid)
    plsc.subcore_barrier()

    def chunk(c, _):
        pltpu.async_copy(hws_sh.at[src_v.at[c]], rows_v, g0).wait()
        pltpu.sync_copy(rows_v, acc_sh.at[dst_v.at[c]], add=True)
        return 0
    lax.fori_loop(0, CPT, chunk, 0)

    @pl.when(wid < NEXTRA)
    def _gextra():
        pltpu.async_copy(hws_sh.at[src_v.at[CPT]], rows_v, g0).wait()
        pltpu.sync_copy(rows_v, acc_sh.at[dst_v.at[CPT]], add=True)
    plsc.subcore_barrier()

    pltpu.sync_copy(acc_sh.at[pl.ds(base, RPT)],
                    out.at[cid, pl.ds(base, RPT)])


_agg2_call = pl.kernel(
    _agg2_body,
    out_type=jax.ShapeDtypeStruct((NC, N, D_H), jnp.float32),
    mesh=_mesh,
    compiler_params=_sc_params,
    scratch_types=[
        pltpu.VMEM((CPT + 1, B), jnp.int32),
        pltpu.VMEM((CPT + 1, B), jnp.int32),
        pltpu.VMEM((B, D_H), jnp.float32),
        pltpu.VMEM((ZB, D_H), jnp.float32),
        pltpu.VMEM_SHARED((N, D_H), jnp.float32),
        pltpu.VMEM_SHARED((N, D_H), jnp.float32),
        pltpu.SemaphoreType.DMA,
    ],
)


_RB = 1000


def _mm1_body(x_ref, w_ref, hw_ref):
    hw_ref[...] = jnp.dot(x_ref[...], w_ref[...],
                          preferred_element_type=jnp.float32)


def _mm1(x, W1):
    return pl.pallas_call(
        _mm1_body,
        grid=(N // _RB,),
        in_specs=[
            pl.BlockSpec((_RB, D_IN), lambda i: (i, 0)),
            pl.BlockSpec((D_IN, D_H), lambda i: (0, 0)),
        ],
        out_specs=pl.BlockSpec((_RB, D_H), lambda i: (i, 0)),
        out_shape=jax.ShapeDtypeStruct((N, D_H), jnp.float32),
    )(x, W1)


def _mid_body(agg_ref, deg_ref, hw1_ref, b1_ref, w2_ref, hws2_ref, dinv_ref):
    deg = deg_ref[0][:, 0:1] + deg_ref[1][:, 0:1] + 1.0
    dinv = lax.rsqrt(deg)
    dinv_ref[...] = dinv
    s = agg_ref[0] + agg_ref[1] + hw1_ref[...] * dinv
    h = jnp.maximum(s * dinv + b1_ref[...], 0.0)
    hw2 = jnp.dot(h, w2_ref[...], preferred_element_type=jnp.float32)
    hws2_ref[...] = hw2 * dinv


def _mid(agg1, degp, hw1, b1, W2):
    return pl.pallas_call(
        _mid_body,
        grid=(N // _RB,),
        in_specs=[
            pl.BlockSpec((NC, _RB, D_H), lambda i: (0, i, 0)),
            pl.BlockSpec((NC, _RB, DW), lambda i: (0, i, 0)),
            pl.BlockSpec((_RB, D_H), lambda i: (i, 0)),
            pl.BlockSpec((1, D_H), lambda i: (0, 0)),
            pl.BlockSpec((D_H, D_H), lambda i: (0, 0)),
        ],
        out_specs=[
            pl.BlockSpec((_RB, D_H), lambda i: (i, 0)),
            pl.BlockSpec((_RB, 1), lambda i: (i, 0)),
        ],
        out_shape=[
            jax.ShapeDtypeStruct((N, D_H), jnp.float32),
            jax.ShapeDtypeStruct((N, 1), jnp.float32),
        ],
    )(agg1, degp, hw1, b1, W2)


def _fin_body(agg_ref, hws2_ref, dinv_ref, b2_ref, wc1_ref, bc1_ref,
              wc2_ref, bc2_ref, emb_ref, probs_ref):
    s = agg_ref[0] + agg_ref[1] + hws2_ref[...]
    emb = jnp.maximum(s * dinv_ref[...] + b2_ref[...], 0.0)
    emb_ref[...] = emb
    z = jnp.maximum(
        jnp.dot(emb, wc1_ref[...], preferred_element_type=jnp.float32)
        + bc1_ref[...], 0.0)
    logits = (jnp.dot(z, wc2_ref[...], preferred_element_type=jnp.float32)
              + bc2_ref[...])
    m = jnp.max(logits, axis=1, keepdims=True)
    e = jnp.exp(logits - m)
    probs_ref[...] = e / jnp.sum(e, axis=1, keepdims=True)


def _fin(agg2, hws2, dinv, b2, Wc1, bc1, Wc2, bc2):
    return pl.pallas_call(
        _fin_body,
        grid=(N // _RB,),
        in_specs=[
            pl.BlockSpec((NC, _RB, D_H), lambda i: (0, i, 0)),
            pl.BlockSpec((_RB, D_H), lambda i: (i, 0)),
            pl.BlockSpec((_RB, 1), lambda i: (i, 0)),
            pl.BlockSpec((1, D_H), lambda i: (0, 0)),
            pl.BlockSpec((D_H, D_H // 2), lambda i: (0, 0)),
            pl.BlockSpec((1, D_H // 2), lambda i: (0, 0)),
            pl.BlockSpec((D_H // 2, N_COMM), lambda i: (0, 0)),
            pl.BlockSpec((1, N_COMM), lambda i: (0, 0)),
        ],
        out_specs=[
            pl.BlockSpec((_RB, D_H), lambda i: (i, 0)),
            pl.BlockSpec((_RB, N_COMM), lambda i: (i, 0)),
        ],
        out_shape=[
            jax.ShapeDtypeStruct((N, D_H), jnp.float32),
            jax.ShapeDtypeStruct((N, N_COMM), jnp.float32),
        ],
    )(agg2, hws2, dinv, b2, Wc1, bc1, Wc2, bc2)


def kernel(x, edge_index, W1, b1, W2, b2, Wc1, bc1, Wc2, bc2):
    srcI = edge_index[0].reshape(NCHUNK, B)
    dstI = edge_index[1].reshape(NCHUNK, B)

    b1r = b1.reshape(1, D_H)
    b2r = b2.reshape(1, D_H)
    bc1r = bc1.reshape(1, D_H // 2)
    bc2r = bc2.reshape(1, N_COMM)

    hw1 = _mm1(x, W1)
    agg1, degp = _agg1_call(hw1, srcI, dstI)
    hws2, dinv = _mid(agg1, degp, hw1, b1r, W2)
    agg2 = _agg2_call(hws2, srcI, dstI)
    emb, probs = _fin(agg2, hws2, dinv, b2r, Wc1, bc1r, Wc2, bc2r)
    return emb, probs


# speedup vs baseline: 1.0700x; 1.0700x over previous
import functools

import jax
import jax.numpy as jnp
from jax import lax
from jax.experimental import pallas as pl
from jax.experimental.pallas import tpu as pltpu
from jax.experimental.pallas import tpu_sc as plsc

N = 10000
E = 320000
D_IN = 128
D_H = 64
N_COMM = 10

NC = 2
NS = 16
NW = NC * NS

B = 128
CH = 80
EPT = CH * B
ET = NW * EPT

KB = 1
NPAD = 10240
RPT = NPAD // NS
DW = 16

_mesh = plsc.VectorSubcoreMesh(core_axis_name="c", subcore_axis_name="s")
_sc_params = pltpu.CompilerParams(use_tc_tiling_on_sc=False)


def _deg_body(dstI, out, dst_v, ones_v, zb_v, acc_sh):
    cid = lax.axis_index("c")
    sid = lax.axis_index("s")

    def fill(i, _):
        ones_v[i, :] = jnp.where(lax.iota(jnp.int32, 16) == 0,
                                 jnp.float32(1), jnp.float32(0))
        return 0
    lax.fori_loop(0, KB * B, fill, 0)

    def zfill(i, _):
        zb_v[i, :] = jnp.zeros((16,), jnp.float32)
        return 0
    lax.fori_loop(0, RPT, zfill, 0)
    pltpu.sync_copy(zb_v, acc_sh.at[pl.ds(sid * RPT, RPT)])
    pltpu.sync_copy(dstI.at[cid, sid], dst_v)
    plsc.subcore_barrier()

    def chunk(c, _):
        pltpu.sync_copy(ones_v, acc_sh.at[dst_v.at[c]], add=True)
        return 0
    lax.fori_loop(0, CH // KB, chunk, 0)
    plsc.subcore_barrier()
    pltpu.sync_copy(acc_sh.at[pl.ds(sid * RPT, RPT)],
                    out.at[cid, pl.ds(sid * RPT, RPT)])


_deg_call = pl.kernel(
    _deg_body,
    out_type=jax.ShapeDtypeStruct((NC, NPAD, DW), jnp.float32),
    mesh=_mesh,
    compiler_params=_sc_params,
    scratch_types=[
        pltpu.VMEM((CH // KB, KB * B), jnp.int32),
        pltpu.VMEM((KB * B, DW), jnp.float32),
        pltpu.VMEM((RPT, DW), jnp.float32),
        pltpu.VMEM_SHARED((NPAD, DW), jnp.float32),
    ],
)


def _agg_body(hws, srcI, dstI, out, src_v, dst_v, rows0, rows1, zb_v, acc_sh,
              hws_sh, g0, g1, s0, s1):
    cid = lax.axis_index("c")
    sid = lax.axis_index("s")

    def zfill(i, _):
        for j in range(D_H // 16):
            zb_v[i, pl.ds(j * 16, 16)] = jnp.zeros((16,), jnp.float32)
        return 0
    lax.fori_loop(0, RPT // 8, zfill, 0)
    for q in range(8):
        pltpu.sync_copy(zb_v, acc_sh.at[pl.ds(sid * RPT + q * (RPT // 8),
                                              RPT // 8)])
    pltpu.sync_copy(hws.at[pl.ds(sid * RPT, RPT)],
                    hws_sh.at[pl.ds(sid * RPT, RPT)])
    pltpu.sync_copy(srcI.at[cid, sid], src_v)
    pltpu.sync_copy(dstI.at[cid, sid], dst_v)
    plsc.subcore_barrier()

    def pair(i, _):
        c2 = i * 2
        hA = pltpu.async_copy(hws_sh.at[src_v.at[c2]], rows0, g0)
        hB = pltpu.async_copy(hws_sh.at[src_v.at[c2 + 1]], rows1, g1)
        hA.wait()
        sA = pltpu.async_copy(rows0, acc_sh.at[dst_v.at[c2]], s0, add=True)
        hB.wait()
        sB = pltpu.async_copy(rows1, acc_sh.at[dst_v.at[c2 + 1]], s1, add=True)
        sA.wait()
        sB.wait()
        return 0
    lax.fori_loop(0, (CH // KB) // 2, pair, 0)
    plsc.subcore_barrier()
    pltpu.sync_copy(acc_sh.at[pl.ds(sid * RPT, RPT)],
                    out.at[cid, pl.ds(sid * RPT, RPT)])


_agg_call = pl.kernel(
    _agg_body,
    out_type=jax.ShapeDtypeStruct((NC, NPAD, D_H), jnp.float32),
    mesh=_mesh,
    compiler_params=_sc_params,
    scratch_types=[
        pltpu.VMEM((CH // KB, KB * B), jnp.int32),
        pltpu.VMEM((CH // KB, KB * B), jnp.int32),
        pltpu.VMEM((KB * B, D_H), jnp.float32),
        pltpu.VMEM((KB * B, D_H), jnp.float32),
        pltpu.VMEM((RPT // 8, D_H), jnp.float32),
        pltpu.VMEM_SHARED((NPAD, D_H), jnp.float32),
        pltpu.VMEM_SHARED((NPAD, D_H), jnp.float32),
        pltpu.SemaphoreType.DMA,
        pltpu.SemaphoreType.DMA,
        pltpu.SemaphoreType.DMA,
        pltpu.SemaphoreType.DMA,
    ],
)


_RB = 1024


def _prep_body(deg_ref, x_ref, w_ref, hws_ref, dinv_ref):
    deg = deg_ref[0][:, 0:1] + deg_ref[1][:, 0:1] + 1.0
    dinv = lax.rsqrt(jnp.maximum(deg, 1.0))
    dinv_ref[...] = dinv
    hw = jnp.dot(x_ref[...], w_ref[...], preferred_element_type=jnp.float32)
    hws_ref[...] = hw * dinv


def _prep(deg, x_pad, W1):
    return pl.pallas_call(
        _prep_body,
        grid=(NPAD // _RB,),
        in_specs=[
            pl.BlockSpec((NC, _RB, DW), lambda i: (0, i, 0)),
            pl.BlockSpec((_RB, D_IN), lambda i: (i, 0)),
            pl.BlockSpec((D_IN, D_H), lambda i: (0, 0)),
        ],
        out_specs=[
            pl.BlockSpec((_RB, D_H), lambda i: (i, 0)),
            pl.BlockSpec((_RB, 1), lambda i: (i, 0)),
        ],
        out_shape=[
            jax.ShapeDtypeStruct((NPAD, D_H), jnp.float32),
            jax.ShapeDtypeStruct((NPAD, 1), jnp.float32),
        ],
    )(deg, x_pad, W1)


def _mid_body(agg_ref, hws1_ref, dinv_ref, b1_ref, w2_ref, hws2_ref):
    s = agg_ref[0] + agg_ref[1] + hws1_ref[...]
    h = jnp.maximum(s * dinv_ref[...] + b1_ref[...], 0.0)
    hw2 = jnp.dot(h, w2_ref[...], preferred_element_type=jnp.float32)
    hws2_ref[...] = hw2 * dinv_ref[...]


def _mid(agg1, hws1, dinv, b1, W2):
    return pl.pallas_call(
        _mid_body,
        grid=(NPAD // _RB,),
        in_specs=[
            pl.BlockSpec((NC, _RB, D_H), lambda i: (0, i, 0)),
            pl.BlockSpec((_RB, D_H), lambda i: (i, 0)),
            pl.BlockSpec((_RB, 1), lambda i: (i, 0)),
            pl.BlockSpec((1, D_H), lambda i: (0, 0)),
            pl.BlockSpec((D_H, D_H), lambda i: (0, 0)),
        ],
        out_specs=pl.BlockSpec((_RB, D_H), lambda i: (i, 0)),
        out_shape=jax.ShapeDtypeStruct((NPAD, D_H), jnp.float32),
    )(agg1, hws1, dinv, b1, W2)


def _fin_body(agg_ref, hws2_ref, dinv_ref, b2_ref, wc1_ref, bc1_ref,
              wc2_ref, bc2_ref, emb_ref, probs_ref):
    s = agg_ref[0] + agg_ref[1] + hws2_ref[...]
    emb = jnp.maximum(s * dinv_ref[...] + b2_ref[...], 0.0)
    emb_ref[...] = emb
    z = jnp.maximum(
        jnp.dot(emb, wc1_ref[...], preferred_element_type=jnp.float32)
        + bc1_ref[...], 0.0)
    logits = (jnp.dot(z, wc2_ref[...], preferred_element_type=jnp.float32)
              + bc2_ref[...])
    m = jnp.max(logits, axis=1, keepdims=True)
    e = jnp.exp(logits - m)
    probs_ref[...] = e / jnp.sum(e, axis=1, keepdims=True)


def _fin(agg2, hws2, dinv, b2, Wc1, bc1, Wc2, bc2):
    return pl.pallas_call(
        _fin_body,
        grid=(NPAD // _RB,),
        in_specs=[
            pl.BlockSpec((NC, _RB, D_H), lambda i: (0, i, 0)),
            pl.BlockSpec((_RB, D_H), lambda i: (i, 0)),
            pl.BlockSpec((_RB, 1), lambda i: (i, 0)),
            pl.BlockSpec((1, D_H), lambda i: (0, 0)),
            pl.BlockSpec((D_H, D_H // 2), lambda i: (0, 0)),
            pl.BlockSpec((1, D_H // 2), lambda i: (0, 0)),
            pl.BlockSpec((D_H // 2, N_COMM), lambda i: (0, 0)),
            pl.BlockSpec((1, N_COMM), lambda i: (0, 0)),
        ],
        out_specs=[
            pl.BlockSpec((_RB, D_H), lambda i: (i, 0)),
            pl.BlockSpec((_RB, N_COMM), lambda i: (i, 0)),
        ],
        out_shape=[
            jax.ShapeDtypeStruct((NPAD, D_H), jnp.float32),
            jax.ShapeDtypeStruct((NPAD, N_COMM), jnp.float32),
        ],
    )(agg2, hws2, dinv, b2, Wc1, bc1, Wc2, bc2)


def kernel(x, edge_index, W1, b1, W2, b2, Wc1, bc1, Wc2, bc2):
    src = edge_index[0]
    dst = edge_index[1]
    pad = ET - E
    src_p = jnp.concatenate([src, jnp.zeros((pad,), jnp.int32)])
    dst_p = jnp.concatenate([dst, jnp.full((pad,), N, jnp.int32)])
    srcI = src_p.reshape(NC, NS, CH // KB, KB * B)
    dstI = dst_p.reshape(NC, NS, CH // KB, KB * B)

    x_pad = jnp.pad(x, ((0, NPAD - N), (0, 0)))
    b1r = b1.reshape(1, D_H)
    b2r = b2.reshape(1, D_H)
    bc1r = bc1.reshape(1, D_H // 2)
    bc2r = bc2.reshape(1, N_COMM)

    deg = _deg_call(dstI)
    hws1, dinv = _prep(deg, x_pad, W1)
    agg1 = _agg_call(hws1, srcI, dstI)
    hws2 = _mid(agg1, hws1, dinv, b1r, W2)
    agg2 = _agg_call(hws2, srcI, dstI)
    emb_pad, probs_pad = _fin(agg2, hws2, dinv, b2r, Wc1, bc1r, Wc2, bc2r)
    return emb_pad[:N], probs_pad[:N]
